# baseline (speedup 1.0000x reference)
import jax
import jax.numpy as jnp
from jax.experimental import pallas as pl

_B, _S, _D = 4, 4096, 4096
_K = 1024
_SB = 512
_CH = 512


def _mv_kernel(x_ref, w_ref, b_ref, o_ref):
    xb = x_ref[0].astype(jnp.bfloat16).astype(jnp.float32)
    wb = w_ref[...].astype(jnp.bfloat16).astype(jnp.float32)
    res = jax.lax.dot_general(
        xb, wb,
        dimension_numbers=(((1,), (1,)), ((), ())),
        preferred_element_type=jnp.float32,
    )
    o_ref[0, 0, :] = res[:, 0] + b_ref[0, 0]


def _topk_kernel(l_ref, o_ref):
    pr = l_ref[0]
    m = jnp.max(pr, axis=1, keepdims=True)
    e = jnp.exp(pr - m)
    arr = e.reshape(1, 32, 128)
    n = 32
    while n > 1:
        arr = arr[:, : n // 2, :] + arr[:, n // 2 :, :]
        n //= 2
    a = arr[:, 0, :]
    n = 128
    while n > 1:
        a = a[:, : n // 2] + a[:, n // 2 :]
        n //= 2
    s = a
    p = e / s

    pt = jnp.transpose(p)
    ranks = jnp.zeros((1, _S), jnp.int32)
    i_lane = jax.lax.broadcasted_iota(jnp.int32, (_CH, _S), 1)
    j_sub = jax.lax.broadcasted_iota(jnp.int32, (_CH, _S), 0)
    for c in range(_S // _CH):
        pj = pt[c * _CH:(c + 1) * _CH, :]
        jj = j_sub + (c * _CH)
        gt = (pj > p).astype(jnp.int32)
        eq = jnp.logical_and(pj == p, jj < i_lane).astype(jnp.int32)
        ranks = ranks + jnp.sum(gt + eq, axis=0, keepdims=True)

    rt = jnp.transpose(ranks)
    r_lane = jax.lax.broadcasted_iota(jnp.int32, (_CH, _K), 1)
    i_sub = jax.lax.broadcasted_iota(jnp.int32, (_CH, _K), 0)
    out = jnp.zeros((1, _K), jnp.int32)
    for c in range(_S // _CH):
        ri = rt[c * _CH:(c + 1) * _CH, :]
        ii = i_sub + (c * _CH)
        hit = (ri == r_lane)
        out = out + jnp.sum(jnp.where(hit, ii, 0), axis=0, keepdims=True)
    o_ref[0] = out


def kernel(inputs, W, b):
    B, S, D = inputs.shape
    b2 = b.reshape(1, 1)
    logits = pl.pallas_call(
        _mv_kernel,
        grid=(B, S // _SB),
        in_specs=[
            pl.BlockSpec((1, _SB, D), lambda i, j: (i, j, 0)),
            pl.BlockSpec((1, D), lambda i, j: (0, 0)),
            pl.BlockSpec((1, 1), lambda i, j: (0, 0)),
        ],
        out_specs=pl.BlockSpec((1, 1, _SB), lambda i, j: (i * (S // _SB) + j, 0, 0)),
        out_shape=jax.ShapeDtypeStruct((B * (S // _SB), 1, _SB), jnp.float32),
    )(inputs, W, b2).reshape(B, 1, S)

    idx = pl.pallas_call(
        _topk_kernel,
        grid=(B,),
        in_specs=[pl.BlockSpec((1, 1, S), lambda i: (i, 0, 0))],
        out_specs=pl.BlockSpec((1, 1, _K), lambda i: (i, 0, 0)),
        out_shape=jax.ShapeDtypeStruct((B, 1, _K), jnp.int32),
    )(logits)
    return idx.reshape(B, _K)


# speedup vs baseline: 1.0619x; 1.0619x over previous
import jax
import jax.numpy as jnp
from jax.experimental import pallas as pl

_B, _S, _D = 4, 4096, 4096
_K = 1024
_SB = 512
_CH = 512


def _mv_kernel(x_ref, w_ref, b_ref, o_ref):
    xb = x_ref[0].astype(jnp.bfloat16).astype(jnp.float32)
    wb = w_ref[...].astype(jnp.bfloat16).astype(jnp.float32)
    res = jax.lax.dot_general(
        xb, wb,
        dimension_numbers=(((1,), (1,)), ((), ())),
        preferred_element_type=jnp.float32,
    )
    o_ref[0, 0, :] = res[:, 0] + b_ref[0, 0]


def _topk_kernel(l_ref, o_ref):
    pr = l_ref[0]
    m = jnp.max(pr, axis=1, keepdims=True)
    e = jnp.exp(pr - m)
    arr = e.reshape(1, 32, 128)
    n = 32
    while n > 1:
        arr = arr[:, : n // 2, :] + arr[:, n // 2 :, :]
        n //= 2
    a = arr[:, 0, :]
    n = 128
    while n > 1:
        a = a[:, : n // 2] + a[:, n // 2 :]
        n //= 2
    s = a
    p = e / s

    pt = jnp.transpose(p)
    ones_row = jnp.ones((1, _CH), jnp.bfloat16)
    rank_f = jnp.zeros((1, _S), jnp.float32)
    i_lane = jax.lax.broadcasted_iota(jnp.int32, (_CH, _S), 1)
    j_sub = jax.lax.broadcasted_iota(jnp.int32, (_CH, _S), 0)
    for c in range(_S // _CH):
        pj = pt[c * _CH:(c + 1) * _CH, :]
        jj = j_sub + (c * _CH)
        sel = jnp.logical_or(pj > p, jnp.logical_and(pj == p, jj < i_lane))
        mat = jnp.where(sel, 1.0, 0.0).astype(jnp.bfloat16)
        rank_f = rank_f + jax.lax.dot_general(
            ones_row, mat, (((1,), (0,)), ((), ())),
            preferred_element_type=jnp.float32)
    ranks = rank_f.astype(jnp.int32)

    rt = jnp.transpose(ranks)
    r_lane = jax.lax.broadcasted_iota(jnp.int32, (_CH, _K), 1)
    i_sub = jax.lax.broadcasted_iota(jnp.int32, (_CH, _K), 0)
    out = jnp.zeros((1, _K), jnp.int32)
    for c in range(_S // _CH):
        ri = rt[c * _CH:(c + 1) * _CH, :]
        ii = i_sub + (c * _CH)
        hit = (ri == r_lane)
        out = out + jnp.sum(jnp.where(hit, ii, 0), axis=0, keepdims=True)
    o_ref[0] = out


def kernel(inputs, W, b):
    B, S, D = inputs.shape
    b2 = b.reshape(1, 1)
    logits = pl.pallas_call(
        _mv_kernel,
        grid=(B, S // _SB),
        in_specs=[
            pl.BlockSpec((1, _SB, D), lambda i, j: (i, j, 0)),
            pl.BlockSpec((1, D), lambda i, j: (0, 0)),
            pl.BlockSpec((1, 1), lambda i, j: (0, 0)),
        ],
        out_specs=pl.BlockSpec((1, 1, _SB), lambda i, j: (i * (S // _SB) + j, 0, 0)),
        out_shape=jax.ShapeDtypeStruct((B * (S // _SB), 1, _SB), jnp.float32),
    )(inputs, W, b2).reshape(B, 1, S)

    idx = pl.pallas_call(
        _topk_kernel,
        grid=(B,),
        in_specs=[pl.BlockSpec((1, 1, S), lambda i: (i, 0, 0))],
        out_specs=pl.BlockSpec((1, 1, _K), lambda i: (i, 0, 0)),
        out_shape=jax.ShapeDtypeStruct((B, 1, _K), jnp.int32),
    )(logits)
    return idx.reshape(B, _K)


# speedup vs baseline: 1.0812x; 1.0182x over previous
import jax
import jax.numpy as jnp
from jax.experimental import pallas as pl

_B, _S, _D = 4, 4096, 4096
_K = 1024
_SB = 1024
_CH = 512


def _mv_kernel(x_ref, w_ref, b_ref, o_ref):
    xb = x_ref[0].astype(jnp.bfloat16).astype(jnp.float32)
    wb = w_ref[...].astype(jnp.bfloat16).astype(jnp.float32)
    res = jax.lax.dot_general(
        xb, wb,
        dimension_numbers=(((1,), (1,)), ((), ())),
        preferred_element_type=jnp.float32,
    )
    o_ref[0, 0, :] = res[:, 0] + b_ref[0, 0]


def _topk_kernel(l_ref, o_ref):
    pr = l_ref[0]
    m = jnp.max(pr, axis=1, keepdims=True)
    e = jnp.exp(pr - m)
    arr = e.reshape(1, 32, 128)
    n = 32
    while n > 1:
        arr = arr[:, : n // 2, :] + arr[:, n // 2 :, :]
        n //= 2
    a = arr[:, 0, :]
    n = 128
    while n > 1:
        a = a[:, : n // 2] + a[:, n // 2 :]
        n //= 2
    s = a
    p = e / s

    pt = jnp.transpose(p)
    ones_row = jnp.ones((1, _CH), jnp.bfloat16)
    rank_f = jnp.zeros((1, _S), jnp.float32)
    i_lane = jax.lax.broadcasted_iota(jnp.int32, (_CH, _S), 1)
    j_sub = jax.lax.broadcasted_iota(jnp.int32, (_CH, _S), 0)
    for c in range(_S // _CH):
        pj = pt[c * _CH:(c + 1) * _CH, :]
        jj = j_sub + (c * _CH)
        sel = jnp.logical_or(pj > p, jnp.logical_and(pj == p, jj < i_lane))
        mat = jnp.where(sel, 1.0, 0.0).astype(jnp.bfloat16)
        rank_f = rank_f + jax.lax.dot_general(
            ones_row, mat, (((1,), (0,)), ((), ())),
            preferred_element_type=jnp.float32)
    ranks = rank_f.astype(jnp.int32)

    rt = jnp.transpose(ranks)
    r_lane = jax.lax.broadcasted_iota(jnp.int32, (_CH, _K), 1)
    i_sub = jax.lax.broadcasted_iota(jnp.int32, (_CH, _K), 0)
    out = jnp.zeros((1, _K), jnp.int32)
    for c in range(_S // _CH):
        ri = rt[c * _CH:(c + 1) * _CH, :]
        ii = i_sub + (c * _CH)
        hit = (ri == r_lane)
        out = out + jnp.sum(jnp.where(hit, ii, 0), axis=0, keepdims=True)
    o_ref[0] = out


def kernel(inputs, W, b):
    B, S, D = inputs.shape
    b2 = b.reshape(1, 1)
    logits = pl.pallas_call(
        _mv_kernel,
        grid=(B, S // _SB),
        in_specs=[
            pl.BlockSpec((1, _SB, D), lambda i, j: (i, j, 0)),
            pl.BlockSpec((1, D), lambda i, j: (0, 0)),
            pl.BlockSpec((1, 1), lambda i, j: (0, 0)),
        ],
        out_specs=pl.BlockSpec((1, 1, _SB), lambda i, j: (i * (S // _SB) + j, 0, 0)),
        out_shape=jax.ShapeDtypeStruct((B * (S // _SB), 1, _SB), jnp.float32),
    )(inputs, W, b2).reshape(B, 1, S)

    idx = pl.pallas_call(
        _topk_kernel,
        grid=(B,),
        in_specs=[pl.BlockSpec((1, 1, S), lambda i: (i, 0, 0))],
        out_specs=pl.BlockSpec((1, 1, _K), lambda i: (i, 0, 0)),
        out_shape=jax.ShapeDtypeStruct((B, 1, _K), jnp.int32),
    )(logits)
    return idx.reshape(B, _K)
